# baseline (speedup 1.0000x reference)
import jax
import jax.numpy as jnp
from jax import lax
from jax.experimental import pallas as pl
from jax.experimental.pallas import tpu as pltpu

_B, _N, _H, _E = 4, 256, 128, 3


def _mp_body(wmsgt_ref, bmsg_ref, wupd_ref, bupd_ref, adj_hbm, ne_hbm,
             er_hbm, out_ref, adj_s, ne_s, er_s, sem_a, sem_n, sem_e):
    f32 = jnp.float32
    cpa = pltpu.make_async_copy(adj_hbm, adj_s, sem_a)
    cpa.start()
    cpn = pltpu.make_async_copy(ne_hbm, ne_s, sem_n)
    cpn.start()
    cpe = [pltpu.make_async_copy(er_hbm.at[pl.ds(2 * g, 2)],
                                 er_s.at[pl.ds(2 * g, 2)], sem_e.at[g])
           for g in range(2)]
    for cp in cpe:
        cp.start()

    wmsgt = wmsgt_ref[...]
    w1t = wmsgt[:_H, :]
    w2t = wmsgt[_H:, :]
    wupd = wupd_ref[...]
    wua = wupd[:, :_H]
    wub = wupd[:, _H:]

    cpa.wait()
    cpn.wait()
    maskf = (adj_s[...] > 0).astype(f32)
    maskt = maskf.T
    ne_all = ne_s[...].reshape(_B * _N, _H)
    pre_all = (jnp.dot(ne_all, w1t, preferred_element_type=f32)
               + bmsg_ref[...])
    pre_cols = jnp.concatenate(
        [pre_all[b * _N:(b + 1) * _N, :] for b in range(_B)], axis=1)
    term1_cols = jnp.dot(maskf, pre_cols,
                         preferred_element_type=f32)

    me_rows = []
    for b in range(_B):
        if b % 2 == 0:
            cpe[b // 2].wait()
        me_rows.append([
            jnp.sum(maskt * er_s[b, c], axis=0, keepdims=True)
            for c in range(_E)])
    me_all = jnp.concatenate(
        [jnp.concatenate([me_rows[b][c] for b in range(_B)], axis=1)
         for c in range(_E)], axis=0)
    term2_stack = lax.dot_general(
        me_all, w2t, (((0,), (0,)), ((), ())),
        preferred_element_type=f32)

    term1_stack = jnp.concatenate(
        [term1_cols[:, b * _H:(b + 1) * _H] for b in range(_B)], axis=0)
    msgs = term1_stack + term2_stack
    h = (lax.dot_general(ne_all, wua, (((1,), (1,)), ((), ())),
                         preferred_element_type=f32)
         + lax.dot_general(msgs, wub, (((1,), (1,)), ((), ())),
                           preferred_element_type=f32)
         + bupd_ref[...])
    out_ref[...] = jnp.maximum(h, 0.0).reshape(_B, _N, _H)


@jax.jit
def _run(node_embeddings, edge_relations, adjacency, W_msg, b_msg, W_upd,
         b_upd):
    B, N, H = node_embeddings.shape
    E = edge_relations.shape[-1]
    ert = jnp.transpose(edge_relations, (0, 3, 1, 2))
    bmsg2 = b_msg.reshape(1, H)
    bupd2 = b_upd.reshape(1, H)
    hbm = pltpu.MemorySpace.HBM
    return pl.pallas_call(
        _mp_body,
        in_specs=[
            pl.BlockSpec((H + E, H), lambda: (0, 0)),
            pl.BlockSpec((1, H), lambda: (0, 0)),
            pl.BlockSpec((H, 2 * H), lambda: (0, 0)),
            pl.BlockSpec((1, H), lambda: (0, 0)),
            pl.BlockSpec(memory_space=hbm),
            pl.BlockSpec(memory_space=hbm),
            pl.BlockSpec(memory_space=hbm),
        ],
        out_specs=pl.BlockSpec((B, N, H), lambda: (0, 0, 0)),
        out_shape=jax.ShapeDtypeStruct((B, N, H), jnp.float32),
        scratch_shapes=[
            pltpu.VMEM((N, N), jnp.int32),
            pltpu.VMEM((B, N, H), jnp.float32),
            pltpu.VMEM((B, E, N, N), jnp.float32),
            pltpu.SemaphoreType.DMA,
            pltpu.SemaphoreType.DMA,
            pltpu.SemaphoreType.DMA((2,)),
        ],
    )(W_msg.T, bmsg2, W_upd, bupd2, adjacency, node_embeddings, ert)


def kernel(node_embeddings, edge_relations, adjacency, W_msg, b_msg, W_upd,
           b_upd):
    return _run(node_embeddings, edge_relations, adjacency, W_msg, b_msg,
                W_upd, b_upd)


# speedup vs baseline: 1.2706x; 1.2706x over previous
import jax
import jax.numpy as jnp
from jax import lax
from jax.experimental import pallas as pl
from jax.experimental.pallas import tpu as pltpu

_B, _N, _H, _E = 4, 256, 128, 3


def _mp_body(wmsgt_hbm, bmsg_hbm, wupd_hbm, bupd_hbm, adj_hbm, ne_hbm,
             er_hbm, out_ref, adj_s, ne_s, er_s, wmsgt_s, bmsg_s, wupd_s,
             bupd_s, sem_a, sem_n, sem_e, sem_w):
    f32 = jnp.float32
    cpa = pltpu.make_async_copy(adj_hbm, adj_s, sem_a)
    cpa.start()
    cpn = pltpu.make_async_copy(ne_hbm, ne_s, sem_n)
    cpn.start()
    cpw = [pltpu.make_async_copy(src_r, dst_r, sem_w.at[i])
           for i, (src_r, dst_r) in enumerate(
               [(wmsgt_hbm, wmsgt_s), (bmsg_hbm, bmsg_s),
                (wupd_hbm, wupd_s), (bupd_hbm, bupd_s)])]
    for cp in cpw:
        cp.start()
    cpe = pltpu.make_async_copy(er_hbm, er_s, sem_e)
    cpe.start()
    for cp in cpw:
        cp.wait()

    wmsgt = wmsgt_s[...]
    w1t = wmsgt[:_H, :]
    w2t = wmsgt[_H:, :]
    wupd = wupd_s[...]
    wua = wupd[:, :_H]
    wub = wupd[:, _H:]

    cpa.wait()
    cpn.wait()
    maskf = (adj_s[...] > 0).astype(f32)
    maskt = maskf.T
    ne_all = ne_s[...].reshape(_B * _N, _H)
    pre_all = (jnp.dot(ne_all, w1t, preferred_element_type=f32)
               + bmsg_s[...])
    pre_cols = jnp.concatenate(
        [pre_all[b * _N:(b + 1) * _N, :] for b in range(_B)], axis=1)
    term1_cols = jnp.dot(maskf, pre_cols,
                         preferred_element_type=f32)

    cpe.wait()
    me_rows = []
    for b in range(_B):
        me_rows.append([
            jnp.sum(maskt * er_s[b, c], axis=0, keepdims=True)
            for c in range(_E)])
    me_all = jnp.concatenate(
        [jnp.concatenate([me_rows[b][c] for b in range(_B)], axis=1)
         for c in range(_E)], axis=0)
    term2_stack = lax.dot_general(
        me_all, w2t, (((0,), (0,)), ((), ())),
        preferred_element_type=f32)

    term1_stack = jnp.concatenate(
        [term1_cols[:, b * _H:(b + 1) * _H] for b in range(_B)], axis=0)
    msgs = term1_stack + term2_stack
    h = (lax.dot_general(ne_all, wua, (((1,), (1,)), ((), ())),
                         preferred_element_type=f32)
         + lax.dot_general(msgs, wub, (((1,), (1,)), ((), ())),
                           preferred_element_type=f32)
         + bupd_s[...])
    out_ref[...] = jnp.maximum(h, 0.0).reshape(_B, _N, _H)


@jax.jit
def _run(node_embeddings, edge_relations, adjacency, W_msg, b_msg, W_upd,
         b_upd):
    B, N, H = node_embeddings.shape
    E = edge_relations.shape[-1]
    ert = jnp.transpose(edge_relations, (0, 3, 1, 2))
    bmsg2 = b_msg.reshape(1, H)
    bupd2 = b_upd.reshape(1, H)
    hbm = pltpu.MemorySpace.HBM
    return pl.pallas_call(
        _mp_body,
        in_specs=[
            pl.BlockSpec(memory_space=hbm),
            pl.BlockSpec(memory_space=hbm),
            pl.BlockSpec(memory_space=hbm),
            pl.BlockSpec(memory_space=hbm),
            pl.BlockSpec(memory_space=hbm),
            pl.BlockSpec(memory_space=hbm),
            pl.BlockSpec(memory_space=hbm),
        ],
        out_specs=pl.BlockSpec((B, N, H), lambda: (0, 0, 0)),
        out_shape=jax.ShapeDtypeStruct((B, N, H), jnp.float32),
        scratch_shapes=[
            pltpu.VMEM((N, N), jnp.int32),
            pltpu.VMEM((B, N, H), jnp.float32),
            pltpu.VMEM((B, E, N, N), jnp.float32),
            pltpu.VMEM((H + E, H), jnp.float32),
            pltpu.VMEM((1, H), jnp.float32),
            pltpu.VMEM((H, 2 * H), jnp.float32),
            pltpu.VMEM((1, H), jnp.float32),
            pltpu.SemaphoreType.DMA,
            pltpu.SemaphoreType.DMA,
            pltpu.SemaphoreType.DMA,
            pltpu.SemaphoreType.DMA((4,)),
        ],
    )(W_msg.T, bmsg2, W_upd, bupd2, adjacency, node_embeddings, ert)


def kernel(node_embeddings, edge_relations, adjacency, W_msg, b_msg, W_upd,
           b_upd):
    return _run(node_embeddings, edge_relations, adjacency, W_msg, b_msg,
                W_upd, b_upd)
